# baseline (speedup 1.0000x reference)
import jax
import jax.numpy as jnp
from jax import lax
from jax.experimental import pallas as pl
from jax.experimental.pallas import tpu as pltpu
from jax.experimental.pallas import tpu_sc as plsc

VOCAB = 1_000_000
EMB = 64
BATCH = 16384
NEG_K = 20

NC = 2
NS = 16
LANES = 16
NW = NC * NS
SPW = BATCH // NW
CHUNK = 32
NCHUNK = SPW // CHUNK
EJ = EMB // LANES
ROW = 2 * EMB

IDX_COLS = 128
CIX_ROWS = BATCH // IDX_COLS
PART_COLS = 128
PART_ROWS = BATCH * LANES // PART_COLS
PROWS_PW = PART_ROWS // NW


TBLK = 16384
TGRID = (VOCAB + TBLK - 1) // TBLK


def _fuse_body(wct_ref, wot_ref, t_ref):
    x = jnp.concatenate([wct_ref[...], wot_ref[...]], axis=0)
    t_ref[...] = jnp.transpose(x).astype(jnp.bfloat16)


_fuse_call = pl.pallas_call(
    _fuse_body,
    grid=(TGRID,),
    in_specs=[pl.BlockSpec((EMB, TBLK), lambda i: (0, i)),
              pl.BlockSpec((EMB, TBLK), lambda i: (0, i))],
    out_shape=jax.ShapeDtypeStruct((VOCAB, ROW), jnp.bfloat16),
    out_specs=pl.BlockSpec((TBLK, ROW), lambda i: (i, 0)),
)


def _sc_body(cix_hbm, oix_hbm, nix_hbm, t_hbm,
             up_hbm, np_hbm,
             cidx_v, oidx_v, nidx_v, ctr_rows, out_rows, neg_rows,
             up_v, np_v, sem):
    wid = lax.axis_index("s") * NC + lax.axis_index("c")
    irow = wid * (SPW // IDX_COLS)

    pltpu.sync_copy(cix_hbm.at[pl.ds(irow, SPW // IDX_COLS)], cidx_v)
    pltpu.sync_copy(oix_hbm.at[pl.ds(irow, SPW // IDX_COLS)], oidx_v)
    for k in range(NEG_K):
        pltpu.sync_copy(nix_hbm.at[k, pl.ds(irow, SPW // IDX_COLS)],
                        nidx_v.at[k])

    def chunk_body(c, _):
        cb = c * CHUNK
        r = lax.shift_right_logical(c, 2)
        col = lax.mul(lax.rem(c, 4), CHUNK)
        d_ctr = pltpu.make_async_copy(
            t_hbm.at[cidx_v.at[r, pl.ds(col, CHUNK)]], ctr_rows, sem)
        d_out = pltpu.make_async_copy(
            t_hbm.at[oidx_v.at[r, pl.ds(col, CHUNK)]], out_rows, sem)
        d_ctr.start()
        d_out.start()
        d_negs = []
        for k in range(NEG_K):
            d = pltpu.make_async_copy(
                t_hbm.at[nidx_v.at[k, r, pl.ds(col, CHUNK)]],
                neg_rows.at[pl.ds(k * CHUNK, CHUNK)], sem)
            d.start()
            d_negs.append(d)
        d_ctr.wait()
        d_out.wait()
        for d in d_negs:
            d.wait()

        def bf16_half(ref, r, base):
            out = []
            for h in range(2):
                w = plsc.bitcast(ref[r, pl.ds(base + 32 * h, 32)],
                                 jnp.int32)
                out.append(plsc.bitcast(w << 16, jnp.float32))
                out.append(plsc.bitcast(w & jnp.int32(-65536), jnp.float32))
            return out

        def sample_body(sl, _):
            cj = bf16_half(ctr_rows, sl, 0)
            oj = bf16_half(out_rows, sl, EMB)
            u = cj[0] * oj[0]
            for j in range(1, EJ):
                u = u + cj[j] * oj[j]
            nj = bf16_half(neg_rows, sl, EMB)
            acc = [cj[j] * nj[j] for j in range(EJ)]
            for k in range(1, NEG_K):
                nj = bf16_half(neg_rows, k * CHUNK + sl, EMB)
                for j in range(EJ):
                    acc[j] = acc[j] + cj[j] * nj[j]
            nv = (acc[0] + acc[1]) + (acc[2] + acc[3])
            s = cb + sl
            pr = lax.shift_right_logical(s, 3)
            pc = lax.mul(lax.rem(s, 8), LANES)
            up_v[pr, pl.ds(pc, LANES)] = u
            np_v[pr, pl.ds(pc, LANES)] = nv
            return 0

        lax.fori_loop(0, CHUNK, sample_body, 0)
        return 0

    lax.fori_loop(0, NCHUNK, chunk_body, 0)

    pltpu.sync_copy(up_v, up_hbm.at[pl.ds(wid * PROWS_PW, PROWS_PW)])
    pltpu.sync_copy(np_v, np_hbm.at[pl.ds(wid * PROWS_PW, PROWS_PW)])


_sc_call = pl.kernel(
    _sc_body,
    out_type=(jax.ShapeDtypeStruct((PART_ROWS, PART_COLS), jnp.float32),
              jax.ShapeDtypeStruct((PART_ROWS, PART_COLS), jnp.float32)),
    mesh=plsc.VectorSubcoreMesh(core_axis_name="c", subcore_axis_name="s",
                                num_cores=NC, num_subcores=NS),
    scratch_types=[
        pltpu.VMEM((SPW // IDX_COLS, IDX_COLS), jnp.int32),
        pltpu.VMEM((SPW // IDX_COLS, IDX_COLS), jnp.int32),
        pltpu.VMEM((NEG_K, SPW // IDX_COLS, IDX_COLS), jnp.int32),
        pltpu.VMEM((CHUNK, ROW), jnp.bfloat16),
        pltpu.VMEM((CHUNK, ROW), jnp.bfloat16),
        pltpu.VMEM((NEG_K * CHUNK, ROW), jnp.bfloat16),
        pltpu.VMEM((PROWS_PW, PART_COLS), jnp.float32),
        pltpu.VMEM((PROWS_PW, PART_COLS), jnp.float32),
        pltpu.SemaphoreType.DMA,
    ],
    compiler_params=pltpu.CompilerParams(use_tc_tiling_on_sc=False,
                                         needs_layout_passes=False),
)


def _log_sigmoid(x):
    return jnp.minimum(x, 0.0) - jnp.log1p(jnp.exp(-jnp.abs(x)))


def _loss_body(u_ref, n_ref, o_ref):
    sel = jnp.float32(1.0) * (
        lax.broadcasted_iota(jnp.int32, (PART_COLS, PART_COLS // LANES), 0)
        // LANES
        == lax.broadcasted_iota(jnp.int32, (PART_COLS, PART_COLS // LANES), 1))
    dn = (((1,), (0,)), ((), ()))
    u = lax.dot_general(u_ref[...], sel, dn,
                        preferred_element_type=jnp.float32)
    n = lax.dot_general(n_ref[...], sel, dn,
                        preferred_element_type=jnp.float32)
    o_ref[0, 0] = -jnp.sum(_log_sigmoid(u) + _log_sigmoid(-n)) / BATCH


_loss_call = pl.pallas_call(
    _loss_body,
    out_shape=jax.ShapeDtypeStruct((1, 1), jnp.float32),
    out_specs=pl.BlockSpec(memory_space=pltpu.SMEM),
)


def kernel(center, outside, negative, W_center, W_outside):
    t_fused = _fuse_call(W_center.T, W_outside.T)
    cix = center.T.reshape(CIX_ROWS, IDX_COLS)
    oix = outside.T.reshape(CIX_ROWS, IDX_COLS)
    nix = negative.T.reshape(NEG_K, CIX_ROWS, IDX_COLS)
    up, npart = _sc_call(cix, oix, nix, t_fused)
    out = _loss_call(up, npart)
    return out[0, 0]


# speedup vs baseline: 1.0163x; 1.0163x over previous
import jax
import jax.numpy as jnp
from jax import lax
from jax.experimental import pallas as pl
from jax.experimental.pallas import tpu as pltpu
from jax.experimental.pallas import tpu_sc as plsc

VOCAB = 1_000_000
EMB = 64
BATCH = 16384
NEG_K = 20

NC = 2
NS = 16
LANES = 16
NW = NC * NS
SPW = BATCH // NW
CHUNK = 32
NCHUNK = SPW // CHUNK
EJ = EMB // LANES
ROW = 2 * EMB

IDX_COLS = 128
CIX_ROWS = BATCH // IDX_COLS
PART_COLS = 128
PLANES = 32
PART_ROWS = BATCH * PLANES // PART_COLS
PROWS_PW = PART_ROWS // NW


TBLK = 16384
TGRID = (VOCAB + TBLK - 1) // TBLK


def _fuse_body(wct_ref, wot_ref, t_ref):
    x = jnp.concatenate([wct_ref[...], wot_ref[...]], axis=0)
    t_ref[...] = jnp.transpose(x).astype(jnp.bfloat16)


_fuse_call = pl.pallas_call(
    _fuse_body,
    grid=(TGRID,),
    in_specs=[pl.BlockSpec((EMB, TBLK), lambda i: (0, i)),
              pl.BlockSpec((EMB, TBLK), lambda i: (0, i))],
    out_shape=jax.ShapeDtypeStruct((VOCAB, ROW), jnp.bfloat16),
    out_specs=pl.BlockSpec((TBLK, ROW), lambda i: (i, 0)),
)


def _sc_body(cix_hbm, oix_hbm, nix_hbm, t_hbm,
             up_hbm, np_hbm,
             cidx_v, oidx_v, nidx_v, ctr_rows, out_rows, neg_rows,
             up_v, np_v, sem):
    wid = lax.axis_index("s") * NC + lax.axis_index("c")
    irow = wid * (SPW // IDX_COLS)

    pltpu.sync_copy(cix_hbm.at[pl.ds(irow, SPW // IDX_COLS)], cidx_v)
    pltpu.sync_copy(oix_hbm.at[pl.ds(irow, SPW // IDX_COLS)], oidx_v)
    for k in range(NEG_K):
        pltpu.sync_copy(nix_hbm.at[k, pl.ds(irow, SPW // IDX_COLS)],
                        nidx_v.at[k])

    def chunk_body(c, _):
        cb = c * CHUNK
        r = lax.shift_right_logical(c, 2)
        col = lax.mul(lax.rem(c, 4), CHUNK)
        d_ctr = pltpu.make_async_copy(
            t_hbm.at[cidx_v.at[r, pl.ds(col, CHUNK)]], ctr_rows, sem)
        d_out = pltpu.make_async_copy(
            t_hbm.at[oidx_v.at[r, pl.ds(col, CHUNK)]], out_rows, sem)
        d_ctr.start()
        d_out.start()
        d_negs = []
        for k in range(NEG_K):
            d = pltpu.make_async_copy(
                t_hbm.at[nidx_v.at[k, r, pl.ds(col, CHUNK)]],
                neg_rows.at[pl.ds(k * CHUNK, CHUNK)], sem)
            d.start()
            d_negs.append(d)
        d_ctr.wait()
        d_out.wait()
        for d in d_negs:
            d.wait()

        def sample_body(sl, _):
            cj = [ctr_rows[sl, pl.ds(32 * j, 32)] for j in range(2)]
            oj = [out_rows[sl, pl.ds(EMB + 32 * j, 32)] for j in range(2)]
            u = cj[0] * oj[0] + cj[1] * oj[1]
            acc = [cj[j] * neg_rows[sl, pl.ds(EMB + 32 * j, 32)]
                   for j in range(2)]
            for k in range(1, NEG_K):
                for j in range(2):
                    acc[j] = acc[j] + cj[j] * neg_rows[k * CHUNK + sl,
                                                       pl.ds(EMB + 32 * j, 32)]
            nv = acc[0] + acc[1]
            s = cb + sl
            pr = lax.shift_right_logical(s, 2)
            pc = lax.mul(lax.rem(s, 4), PLANES)
            up_v[pr, pl.ds(pc, PLANES)] = u
            np_v[pr, pl.ds(pc, PLANES)] = nv
            return 0

        lax.fori_loop(0, CHUNK, sample_body, 0)
        return 0

    lax.fori_loop(0, NCHUNK, chunk_body, 0)

    pltpu.sync_copy(up_v, up_hbm.at[pl.ds(wid * PROWS_PW, PROWS_PW)])
    pltpu.sync_copy(np_v, np_hbm.at[pl.ds(wid * PROWS_PW, PROWS_PW)])


_sc_call = pl.kernel(
    _sc_body,
    out_type=(jax.ShapeDtypeStruct((PART_ROWS, PART_COLS), jnp.bfloat16),
              jax.ShapeDtypeStruct((PART_ROWS, PART_COLS), jnp.bfloat16)),
    mesh=plsc.VectorSubcoreMesh(core_axis_name="c", subcore_axis_name="s",
                                num_cores=NC, num_subcores=NS),
    scratch_types=[
        pltpu.VMEM((SPW // IDX_COLS, IDX_COLS), jnp.int32),
        pltpu.VMEM((SPW // IDX_COLS, IDX_COLS), jnp.int32),
        pltpu.VMEM((NEG_K, SPW // IDX_COLS, IDX_COLS), jnp.int32),
        pltpu.VMEM((CHUNK, ROW), jnp.bfloat16),
        pltpu.VMEM((CHUNK, ROW), jnp.bfloat16),
        pltpu.VMEM((NEG_K * CHUNK, ROW), jnp.bfloat16),
        pltpu.VMEM((PROWS_PW, PART_COLS), jnp.bfloat16),
        pltpu.VMEM((PROWS_PW, PART_COLS), jnp.bfloat16),
        pltpu.SemaphoreType.DMA,
    ],
    compiler_params=pltpu.CompilerParams(use_tc_tiling_on_sc=False),
)


def _log_sigmoid(x):
    return jnp.minimum(x, 0.0) - jnp.log1p(jnp.exp(-jnp.abs(x)))


def _loss_body(u_ref, n_ref, o_ref):
    sel = jnp.float32(1.0) * (
        lax.broadcasted_iota(jnp.int32, (PART_COLS, PART_COLS // PLANES), 0)
        // PLANES
        == lax.broadcasted_iota(jnp.int32, (PART_COLS, PART_COLS // PLANES),
                                1))
    dn = (((1,), (0,)), ((), ()))
    u = lax.dot_general(u_ref[...].astype(jnp.float32), sel, dn,
                        preferred_element_type=jnp.float32)
    n = lax.dot_general(n_ref[...].astype(jnp.float32), sel, dn,
                        preferred_element_type=jnp.float32)
    o_ref[0, 0] = -jnp.sum(_log_sigmoid(u) + _log_sigmoid(-n)) / BATCH


_loss_call = pl.pallas_call(
    _loss_body,
    out_shape=jax.ShapeDtypeStruct((1, 1), jnp.float32),
    out_specs=pl.BlockSpec(memory_space=pltpu.SMEM),
)


def kernel(center, outside, negative, W_center, W_outside):
    t_fused = _fuse_call(W_center.T, W_outside.T)
    cix = center.T.reshape(CIX_ROWS, IDX_COLS)
    oix = outside.T.reshape(CIX_ROWS, IDX_COLS)
    nix = negative.T.reshape(NEG_K, CIX_ROWS, IDX_COLS)
    up, npart = _sc_call(cix, oix, nix, t_fused)
    out = _loss_call(up, npart)
    return out[0, 0]


# speedup vs baseline: 2.6441x; 2.6017x over previous
import jax
import jax.numpy as jnp
from jax import lax
from jax.experimental import pallas as pl
from jax.experimental.pallas import tpu as pltpu
from jax.experimental.pallas import tpu_sc as plsc

VOCAB = 1_000_000
EMB = 64
BATCH = 16384
NEG_K = 20

NC = 2
NS = 16
LANES = 16
NW = NC * NS
SPW = BATCH // NW
CHUNK = 32
NCHUNK = SPW // CHUNK
EJ = EMB // LANES
ROW = 2 * EMB

IDX_COLS = 128
CIX_ROWS = BATCH // IDX_COLS
PART_COLS = 128
PART_ROWS = BATCH * LANES // PART_COLS
PROWS_PW = PART_ROWS // NW


TBLK = 16384
TGRID = (VOCAB + TBLK - 1) // TBLK


def _fuse_body(wct_ref, wot_ref, t_ref):
    x = jnp.concatenate([wct_ref[...], wot_ref[...]], axis=0)
    t_ref[...] = jnp.transpose(x)


_fuse_call = pl.pallas_call(
    _fuse_body,
    grid=(TGRID,),
    in_specs=[pl.BlockSpec((EMB, TBLK), lambda i: (0, i)),
              pl.BlockSpec((EMB, TBLK), lambda i: (0, i))],
    out_shape=jax.ShapeDtypeStruct((VOCAB, ROW), jnp.float32),
    out_specs=pl.BlockSpec((TBLK, ROW), lambda i: (i, 0)),
)


def _sc_body(cix_hbm, oix_hbm, nix_hbm, t_hbm,
             up_hbm, np_hbm,
             cidx_v, oidx_v, nidx_v, ctr_rows, out_rows, neg_rows,
             up_v, np_v, sem):
    wid = lax.axis_index("s") * NC + lax.axis_index("c")
    irow = wid * (SPW // IDX_COLS)

    pltpu.sync_copy(cix_hbm.at[pl.ds(irow, SPW // IDX_COLS)], cidx_v)
    pltpu.sync_copy(oix_hbm.at[pl.ds(irow, SPW // IDX_COLS)], oidx_v)
    for k in range(NEG_K):
        pltpu.sync_copy(nix_hbm.at[k, pl.ds(irow, SPW // IDX_COLS)],
                        nidx_v.at[k])

    def chunk_body(c, _):
        cb = c * CHUNK
        r = lax.shift_right_logical(c, 2)
        col = lax.mul(lax.rem(c, 4), CHUNK)
        d_ctr = pltpu.make_async_copy(
            t_hbm.at[cidx_v.at[r, pl.ds(col, CHUNK)]], ctr_rows, sem)
        d_out = pltpu.make_async_copy(
            t_hbm.at[oidx_v.at[r, pl.ds(col, CHUNK)]], out_rows, sem)
        d_ctr.start()
        d_out.start()
        d_negs = []
        for k in range(NEG_K):
            d = pltpu.make_async_copy(
                t_hbm.at[nidx_v.at[k, r, pl.ds(col, CHUNK)]],
                neg_rows.at[pl.ds(k * CHUNK, CHUNK)], sem)
            d.start()
            d_negs.append(d)
        d_ctr.wait()
        d_out.wait()
        for d in d_negs:
            d.wait()

        def sample_body(sl, _):
            cj = [ctr_rows[sl, pl.ds(16 * j, 16)] for j in range(EJ)]
            oj = [out_rows[sl, pl.ds(EMB + 16 * j, 16)] for j in range(EJ)]
            u = cj[0] * oj[0]
            for j in range(1, EJ):
                u = u + cj[j] * oj[j]
            acc = [cj[j] * neg_rows[sl, pl.ds(EMB + 16 * j, 16)]
                   for j in range(EJ)]
            for k in range(1, NEG_K):
                for j in range(EJ):
                    acc[j] = acc[j] + cj[j] * neg_rows[k * CHUNK + sl,
                                                       pl.ds(EMB + 16 * j, 16)]
            nv = (acc[0] + acc[1]) + (acc[2] + acc[3])
            s = cb + sl
            pr = lax.shift_right_logical(s, 3)
            pc = lax.mul(lax.rem(s, 8), LANES)
            up_v[pr, pl.ds(pc, LANES)] = u
            np_v[pr, pl.ds(pc, LANES)] = nv
            return 0

        lax.fori_loop(0, CHUNK, sample_body, 0)
        return 0

    lax.fori_loop(0, NCHUNK, chunk_body, 0)

    pltpu.sync_copy(up_v, up_hbm.at[pl.ds(wid * PROWS_PW, PROWS_PW)])
    pltpu.sync_copy(np_v, np_hbm.at[pl.ds(wid * PROWS_PW, PROWS_PW)])


_sc_call = pl.kernel(
    _sc_body,
    out_type=(jax.ShapeDtypeStruct((PART_ROWS, PART_COLS), jnp.float32),
              jax.ShapeDtypeStruct((PART_ROWS, PART_COLS), jnp.float32)),
    mesh=plsc.VectorSubcoreMesh(core_axis_name="c", subcore_axis_name="s",
                                num_cores=NC, num_subcores=NS),
    scratch_types=[
        pltpu.VMEM((SPW // IDX_COLS, IDX_COLS), jnp.int32),
        pltpu.VMEM((SPW // IDX_COLS, IDX_COLS), jnp.int32),
        pltpu.VMEM((NEG_K, SPW // IDX_COLS, IDX_COLS), jnp.int32),
        pltpu.VMEM((CHUNK, ROW), jnp.float32),
        pltpu.VMEM((CHUNK, ROW), jnp.float32),
        pltpu.VMEM((NEG_K * CHUNK, ROW), jnp.float32),
        pltpu.VMEM((PROWS_PW, PART_COLS), jnp.float32),
        pltpu.VMEM((PROWS_PW, PART_COLS), jnp.float32),
        pltpu.SemaphoreType.DMA,
    ],
    compiler_params=pltpu.CompilerParams(use_tc_tiling_on_sc=False),
)


def _log_sigmoid(x):
    return jnp.minimum(x, 0.0) - jnp.log1p(jnp.exp(-jnp.abs(x)))


def _loss_body(u_ref, n_ref, o_ref):
    sel = jnp.float32(1.0) * (
        lax.broadcasted_iota(jnp.int32, (PART_COLS, PART_COLS // LANES), 0)
        // LANES
        == lax.broadcasted_iota(jnp.int32, (PART_COLS, PART_COLS // LANES), 1))
    dn = (((1,), (0,)), ((), ()))
    u = lax.dot_general(u_ref[...], sel, dn,
                        preferred_element_type=jnp.float32)
    n = lax.dot_general(n_ref[...], sel, dn,
                        preferred_element_type=jnp.float32)
    o_ref[0, 0] = -jnp.sum(_log_sigmoid(u) + _log_sigmoid(-n)) / BATCH


_loss_call = pl.pallas_call(
    _loss_body,
    out_shape=jax.ShapeDtypeStruct((1, 1), jnp.float32),
    out_specs=pl.BlockSpec(memory_space=pltpu.SMEM),
)


def kernel(center, outside, negative, W_center, W_outside):
    t_fused = _fuse_call(W_center.T, W_outside.T)
    cix = center.T.reshape(CIX_ROWS, IDX_COLS)
    oix = outside.T.reshape(CIX_ROWS, IDX_COLS)
    nix = negative.T.reshape(NEG_K, CIX_ROWS, IDX_COLS)
    up, npart = _sc_call(cix, oix, nix, t_fused)
    out = _loss_call(up, npart)
    return out[0, 0]
